# baseline (speedup 1.0000x reference)
import functools

import jax
import jax.numpy as jnp
from jax import lax
from jax.experimental import pallas as pl
from jax.experimental.pallas import tpu as pltpu
from jax.experimental.pallas import tpu_sc as plsc

NC = 2
NS = 16
CH = 1024


def _zero_acc_slices(zb, accs, sid, slice_words):
    zero16 = jnp.zeros((16,), jnp.float32)

    def _z(i, _):
        zb[pl.ds(i * 16, 16)] = zero16
        return 0

    lax.fori_loop(0, CH // 16, _z, 0)
    nfull = slice_words // CH
    rem = slice_words % CH
    for acc in accs:
        base = sid * slice_words
        for i in range(nfull):
            pltpu.sync_copy(zb, acc.at[pl.ds(base + i * CH, CH)])
        if rem:
            pltpu.sync_copy(zb.at[pl.ds(0, rem)],
                            acc.at[pl.ds(base + nfull * CH, rem)])


def _body(x0_hbm, a0_hbm, src_hbm, dst_hbm, ea_hbm,
          s2_out, ic_out, t_hbm, r_hbm,
          tab, srcb0, srcb1, dstb0, dstb1, dstb2, dstb3,
          eab0, eab1, vb0, cb0, rb0, vb1, cb1, rb1,
          sem_in, sem_sc, sem_r,
          acc_s, acc_c,
          *, nchunks, n, e, slice_words):
    cid = lax.axis_index("c")
    sid = lax.axis_index("s")

    mc = nchunks // NS + jnp.where(sid < nchunks % NS, 1, 0)
    srcs = (srcb0, srcb1)
    dsts = (dstb0, dstb1, dstb2, dstb3)
    eas = (eab0, eab1)
    outs = ((vb0, cb0, rb0), (vb1, cb1, rb1))

    def off(c):
        return (c * NS + sid) * CH

    def fire_in(c, p2, p4, second):
        o = off(c)
        pltpu.async_copy(src_hbm.at[pl.ds(o, CH)], srcs[p2], sem_in)
        pltpu.async_copy(dst_hbm.at[pl.ds(o, CH)], dsts[p4], sem_in)
        src3 = r_hbm if second else ea_hbm
        pltpu.async_copy(src3.at[pl.ds(cid * e + o, CH)], eas[p2], sem_in)

    def wait_in():
        pltpu.make_async_copy(src_hbm.at[pl.ds(0, CH)], srcb0, sem_in).wait()
        pltpu.make_async_copy(dst_hbm.at[pl.ds(0, CH)], dstb0, sem_in).wait()
        pltpu.make_async_copy(ea_hbm.at[pl.ds(0, CH)], eab0, sem_in).wait()

    def wait_out1():
        pltpu.make_async_copy(ea_hbm.at[pl.ds(0, CH)], vb0, sem_sc).wait()
        pltpu.make_async_copy(ea_hbm.at[pl.ds(0, CH)], cb0, sem_sc).wait()
        pltpu.make_async_copy(ea_hbm.at[pl.ds(0, CH)], rb0, sem_r).wait()

    def wait_out2():
        pltpu.make_async_copy(ea_hbm.at[pl.ds(0, CH)], vb0, sem_sc).wait()

    def body1(c, p2, p4):
        sb = srcs[p2]
        db = dsts[p4]
        eb = eas[p2]
        vb, cb, rb = outs[p2]
        wait_in()

        @pl.when(c + 1 < mc)
        def _():
            fire_in(c + 1, 1 - p2, (p4 + 1) % 4, False)

        @pl.when(c >= 2)
        def _():
            wait_out1()

        def _grp(i, _):
            for u in range(8):
                q = i * 8 + u
                s16 = sb[pl.ds(q * 16, 16)]
                d16 = db[pl.ds(q * 16, 16)]
                xs = plsc.load_gather(tab, [s16])
                xd = plsc.load_gather(tab, [d16])
                ea = eb[pl.ds(q * 16, 16)]
                m = ea != 0.0
                cnum = jnp.where(m, 1.0, 0.0)
                rv = cnum / jnp.where(m, ea, 1.0)
                vb[pl.ds(q * 16, 16)] = (xd - xs) * rv
                cb[pl.ds(q * 16, 16)] = cnum
                rb[pl.ds(q * 16, 16)] = rv
            return 0

        lax.fori_loop(0, CH // 128, _grp, 0)
        pltpu.async_copy(vb, acc_s.at[db], sem_sc, add=True)
        pltpu.async_copy(cb, acc_c.at[db], sem_sc, add=True)
        pltpu.async_copy(rb, r_hbm.at[pl.ds(cid * e + off(c), CH)], sem_r)

    def body2(c, p2, p4):
        sb = srcs[p2]
        db = dsts[p4]
        rb_in = eas[p2]
        vb = outs[p2][0]
        wait_in()

        @pl.when(c + 1 < mc)
        def _():
            fire_in(c + 1, 1 - p2, (p4 + 1) % 4, True)

        @pl.when(c >= 2)
        def _():
            wait_out2()

        def _grp(i, _):
            for u in range(8):
                q = i * 8 + u
                s16 = sb[pl.ds(q * 16, 16)]
                d16 = db[pl.ds(q * 16, 16)]
                ts = plsc.load_gather(tab, [s16])
                td = plsc.load_gather(tab, [d16])
                rv = rb_in[pl.ds(q * 16, 16)]
                vb[pl.ds(q * 16, 16)] = (td - ts) * rv
            return 0

        lax.fori_loop(0, CH // 128, _grp, 0)
        pltpu.async_copy(vb, acc_s.at[db], sem_sc, add=True)

    def run_pass(body):
        def quad(c4, _):
            body(4 * c4, 0, 0)
            body(4 * c4 + 1, 1, 1)
            body(4 * c4 + 2, 0, 2)
            body(4 * c4 + 3, 1, 3)
            return 0

        lax.fori_loop(0, mc // 4, quad, 0)
        base = (mc // 4) * 4
        for t in range(3):
            @pl.when(base + t < mc)
            def _(t=t):
                body(base + t, t % 2, t % 4)

    pltpu.sync_copy(x0_hbm, tab.at[pl.ds(0, n)])
    _zero_acc_slices(vb0, (acc_s, acc_c), sid, slice_words)
    plsc.subcore_barrier()
    fire_in(0, 0, 0, False)
    run_pass(body1)
    wait_out1()
    wait_out1()
    plsc.subcore_barrier()

    base = sid * slice_words
    nsub = -(-slice_words // CH)
    for i in range(nsub):
        sz = min(CH, slice_words - i * CH)
        o = base + i * CH
        pltpu.sync_copy(acc_s.at[pl.ds(o, sz)], vb0.at[pl.ds(0, sz)])
        pltpu.sync_copy(acc_c.at[pl.ds(o, sz)], cb0.at[pl.ds(0, sz)])
        pltpu.sync_copy(a0_hbm.at[pl.ds(o, sz)], rb0.at[pl.ds(0, sz)])

        def _t(g, _):
            s = vb0[pl.ds(g * 16, 16)]
            cc = cb0[pl.ds(g * 16, 16)]
            a0 = rb0[pl.ds(g * 16, 16)]
            ic = 1.0 / jnp.maximum(cc, 1.0)
            vb1[pl.ds(g * 16, 16)] = a0 * s * ic
            cb1[pl.ds(g * 16, 16)] = ic
            return 0

        lax.fori_loop(0, sz // 16, _t, 0)
        pltpu.sync_copy(vb1.at[pl.ds(0, sz)], t_hbm.at[cid, pl.ds(o, sz)])
        pltpu.sync_copy(cb1.at[pl.ds(0, sz)], ic_out.at[cid, pl.ds(o, sz)])
    plsc.subcore_barrier()

    pltpu.sync_copy(t_hbm.at[cid], tab)
    _zero_acc_slices(vb0, (acc_s,), sid, slice_words)
    plsc.subcore_barrier()

    fire_in(0, 0, 0, True)
    run_pass(body2)
    wait_out2()
    wait_out2()
    plsc.subcore_barrier()

    sl = slice_words
    pltpu.sync_copy(acc_s.at[pl.ds(sid * sl, sl)],
                    s2_out.at[cid, pl.ds(sid * sl, sl)])


def _combine2_body(s2_ref, ic_ref, o_ref):
    o_ref[...] = s2_ref[0] * ic_ref[0] + s2_ref[1] * ic_ref[1] + 1.0


def kernel(out_x, a_x_x, edge_attr, edge_index):
    n = out_x.shape[0]
    e = edge_index.shape[1]
    nchunks = e // CH
    slice_words = -(-n // (NS * 128)) * 128
    n_pad = NS * slice_words

    src_f = edge_index[0]
    dst_f = edge_index[1]
    ea_f = jnp.concatenate([edge_attr[:, 0], edge_attr[:, 1]])
    x0 = out_x[:, 0]
    a0p = jnp.pad(a_x_x[:, 0], (0, n_pad - n))

    mesh = plsc.VectorSubcoreMesh(
        core_axis_name="c", subcore_axis_name="s",
        num_cores=NC, num_subcores=NS)

    f32 = jnp.float32
    i32 = jnp.int32
    sc_params = pltpu.CompilerParams(needs_layout_passes=False)
    sc_kernel = pl.kernel(
        functools.partial(_body, nchunks=nchunks, n=n, e=e,
                          slice_words=slice_words),
        out_type=(
            jax.ShapeDtypeStruct((NC, n_pad), f32),
            jax.ShapeDtypeStruct((NC, n_pad), f32),
            jax.ShapeDtypeStruct((NC, n_pad), f32),
            jax.ShapeDtypeStruct((NC * e,), f32),
        ),
        mesh=mesh,
        compiler_params=sc_params,
        scratch_types=[
            pltpu.VMEM((n_pad,), f32),
            pltpu.VMEM((CH,), i32),
            pltpu.VMEM((CH,), i32),
            pltpu.VMEM((CH,), i32),
            pltpu.VMEM((CH,), i32),
            pltpu.VMEM((CH,), i32),
            pltpu.VMEM((CH,), i32),
            pltpu.VMEM((CH,), f32),
            pltpu.VMEM((CH,), f32),
            pltpu.VMEM((CH,), f32),
            pltpu.VMEM((CH,), f32),
            pltpu.VMEM((CH,), f32),
            pltpu.VMEM((CH,), f32),
            pltpu.VMEM((CH,), f32),
            pltpu.VMEM((CH,), f32),
            pltpu.SemaphoreType.DMA,
            pltpu.SemaphoreType.DMA,
            pltpu.SemaphoreType.DMA,
            pltpu.VMEM_SHARED((n_pad,), f32),
            pltpu.VMEM_SHARED((n_pad,), f32),
        ],
    )
    s2, ic, _t, _r = sc_kernel(x0, a0p, src_f, dst_f, ea_f)

    tc_rows = n_pad // 128
    out = pl.pallas_call(
        _combine2_body,
        out_shape=jax.ShapeDtypeStruct((tc_rows, 128), f32),
    )(s2.reshape(NC, tc_rows, 128), ic.reshape(NC, tc_rows, 128))
    return out.reshape(n_pad)[:n]


# speedup vs baseline: 1.0063x; 1.0063x over previous
import functools

import jax
import jax.numpy as jnp
from jax import lax
from jax.experimental import pallas as pl
from jax.experimental.pallas import tpu as pltpu
from jax.experimental.pallas import tpu_sc as plsc

NC = 2
NS = 16
CH = 1024


def _zero_acc_slices(zb, accs, sid, slice_words):
    zero16 = jnp.zeros((16,), jnp.float32)

    def _z(i, _):
        zb[pl.ds(i * 16, 16)] = zero16
        return 0

    lax.fori_loop(0, CH // 16, _z, 0)
    nfull = slice_words // CH
    rem = slice_words % CH
    for acc in accs:
        base = sid * slice_words
        for i in range(nfull):
            pltpu.sync_copy(zb, acc.at[pl.ds(base + i * CH, CH)])
        if rem:
            pltpu.sync_copy(zb.at[pl.ds(0, rem)],
                            acc.at[pl.ds(base + nfull * CH, rem)])


def _body(x0_hbm, a0_hbm, src_hbm, dst_hbm, ea_hbm,
          s2_out, ic_out, t_hbm, r_hbm,
          tab, srcb0, srcb1, dstb0, dstb1, dstb2, dstb3,
          eab0, eab1, vb0, cb0, rb0, vb1, cb1, rb1,
          sem_in, sem_sc, sem_r,
          acc_s, acc_c,
          *, nchunks, n, e, slice_words):
    cid = lax.axis_index("c")
    sid = lax.axis_index("s")

    mc = nchunks // NS + jnp.where(sid < nchunks % NS, 1, 0)
    srcs = (srcb0, srcb1)
    dsts = (dstb0, dstb1, dstb2, dstb3)
    eas = (eab0, eab1)
    outs = ((vb0, cb0, rb0), (vb1, cb1, rb1))

    def off(c):
        return (c * NS + sid) * CH

    def fire_in(c, p2, p4, second):
        o = off(c)
        pltpu.async_copy(src_hbm.at[pl.ds(o, CH)], srcs[p2], sem_in)
        pltpu.async_copy(dst_hbm.at[pl.ds(o, CH)], dsts[p4], sem_in)
        src3 = r_hbm if second else ea_hbm
        pltpu.async_copy(src3.at[pl.ds(cid * e + o, CH)], eas[p2], sem_in)

    def wait_in():
        pltpu.make_async_copy(src_hbm.at[pl.ds(0, CH)], srcb0, sem_in).wait()
        pltpu.make_async_copy(dst_hbm.at[pl.ds(0, CH)], dstb0, sem_in).wait()
        pltpu.make_async_copy(ea_hbm.at[pl.ds(0, CH)], eab0, sem_in).wait()

    def wait_out1():
        pltpu.make_async_copy(ea_hbm.at[pl.ds(0, CH)], vb0, sem_sc).wait()
        pltpu.make_async_copy(ea_hbm.at[pl.ds(0, CH)], cb0, sem_sc).wait()
        pltpu.make_async_copy(ea_hbm.at[pl.ds(0, CH)], rb0, sem_r).wait()

    def wait_out2():
        pltpu.make_async_copy(ea_hbm.at[pl.ds(0, CH)], vb0, sem_sc).wait()

    def body1(c, p2, p4):
        sb = srcs[p2]
        db = dsts[p4]
        eb = eas[p2]
        vb, cb, rb = outs[p2]
        wait_in()

        @pl.when(c + 1 < mc)
        def _():
            fire_in(c + 1, 1 - p2, (p4 + 1) % 4, False)

        @pl.when(c >= 2)
        def _():
            wait_out1()

        def _grp(q):
            s16 = sb[pl.ds(q * 16, 16)]
            d16 = db[pl.ds(q * 16, 16)]
            xs = plsc.load_gather(tab, [s16])
            xd = plsc.load_gather(tab, [d16])
            ea = eb[pl.ds(q * 16, 16)]
            m = ea != 0.0
            cnum = jnp.where(m, 1.0, 0.0)
            rv = cnum / jnp.where(m, ea, 1.0)
            vb[pl.ds(q * 16, 16)] = (xd - xs) * rv
            cb[pl.ds(q * 16, 16)] = cnum
            rb[pl.ds(q * 16, 16)] = rv

        plsc.parallel_loop(0, CH // 16, 1, unroll=4)(_grp)
        pltpu.async_copy(vb, acc_s.at[db], sem_sc, add=True)
        pltpu.async_copy(cb, acc_c.at[db], sem_sc, add=True)
        pltpu.async_copy(rb, r_hbm.at[pl.ds(cid * e + off(c), CH)], sem_r)

    def body2(c, p2, p4):
        sb = srcs[p2]
        db = dsts[p4]
        rb_in = eas[p2]
        vb = outs[p2][0]
        wait_in()

        @pl.when(c + 1 < mc)
        def _():
            fire_in(c + 1, 1 - p2, (p4 + 1) % 4, True)

        @pl.when(c >= 2)
        def _():
            wait_out2()

        def _grp(q):
            s16 = sb[pl.ds(q * 16, 16)]
            d16 = db[pl.ds(q * 16, 16)]
            ts = plsc.load_gather(tab, [s16])
            td = plsc.load_gather(tab, [d16])
            rv = rb_in[pl.ds(q * 16, 16)]
            vb[pl.ds(q * 16, 16)] = (td - ts) * rv

        plsc.parallel_loop(0, CH // 16, 1, unroll=4)(_grp)
        pltpu.async_copy(vb, acc_s.at[db], sem_sc, add=True)

    def run_pass(body):
        def quad(c4, _):
            body(4 * c4, 0, 0)
            body(4 * c4 + 1, 1, 1)
            body(4 * c4 + 2, 0, 2)
            body(4 * c4 + 3, 1, 3)
            return 0

        lax.fori_loop(0, mc // 4, quad, 0)
        base = (mc // 4) * 4
        for t in range(3):
            @pl.when(base + t < mc)
            def _(t=t):
                body(base + t, t % 2, t % 4)

    pltpu.sync_copy(x0_hbm, tab.at[pl.ds(0, n)])
    _zero_acc_slices(vb0, (acc_s, acc_c), sid, slice_words)
    plsc.subcore_barrier()
    fire_in(0, 0, 0, False)
    run_pass(body1)
    wait_out1()
    wait_out1()
    plsc.subcore_barrier()

    base = sid * slice_words
    nsub = -(-slice_words // CH)
    for i in range(nsub):
        sz = min(CH, slice_words - i * CH)
        o = base + i * CH
        pltpu.sync_copy(acc_s.at[pl.ds(o, sz)], vb0.at[pl.ds(0, sz)])
        pltpu.sync_copy(acc_c.at[pl.ds(o, sz)], cb0.at[pl.ds(0, sz)])
        pltpu.sync_copy(a0_hbm.at[pl.ds(o, sz)], rb0.at[pl.ds(0, sz)])

        def _t(g, _):
            s = vb0[pl.ds(g * 16, 16)]
            cc = cb0[pl.ds(g * 16, 16)]
            a0 = rb0[pl.ds(g * 16, 16)]
            ic = 1.0 / jnp.maximum(cc, 1.0)
            vb1[pl.ds(g * 16, 16)] = a0 * s * ic
            cb1[pl.ds(g * 16, 16)] = ic
            return 0

        lax.fori_loop(0, sz // 16, _t, 0)
        pltpu.sync_copy(vb1.at[pl.ds(0, sz)], t_hbm.at[cid, pl.ds(o, sz)])
        pltpu.sync_copy(cb1.at[pl.ds(0, sz)], ic_out.at[cid, pl.ds(o, sz)])
    plsc.subcore_barrier()

    pltpu.sync_copy(t_hbm.at[cid], tab)
    _zero_acc_slices(vb0, (acc_s,), sid, slice_words)
    plsc.subcore_barrier()

    fire_in(0, 0, 0, True)
    run_pass(body2)
    wait_out2()
    wait_out2()
    plsc.subcore_barrier()

    sl = slice_words
    pltpu.sync_copy(acc_s.at[pl.ds(sid * sl, sl)],
                    s2_out.at[cid, pl.ds(sid * sl, sl)])


def _combine2_body(s2_ref, ic_ref, o_ref):
    o_ref[...] = s2_ref[0] * ic_ref[0] + s2_ref[1] * ic_ref[1] + 1.0


def kernel(out_x, a_x_x, edge_attr, edge_index):
    n = out_x.shape[0]
    e = edge_index.shape[1]
    nchunks = e // CH
    slice_words = -(-n // (NS * 128)) * 128
    n_pad = NS * slice_words

    src_f = edge_index[0]
    dst_f = edge_index[1]
    ea_f = jnp.concatenate([edge_attr[:, 0], edge_attr[:, 1]])
    x0 = out_x[:, 0]
    a0p = jnp.pad(a_x_x[:, 0], (0, n_pad - n))

    mesh = plsc.VectorSubcoreMesh(
        core_axis_name="c", subcore_axis_name="s",
        num_cores=NC, num_subcores=NS)

    f32 = jnp.float32
    i32 = jnp.int32
    sc_params = pltpu.CompilerParams(needs_layout_passes=False)
    sc_kernel = pl.kernel(
        functools.partial(_body, nchunks=nchunks, n=n, e=e,
                          slice_words=slice_words),
        out_type=(
            jax.ShapeDtypeStruct((NC, n_pad), f32),
            jax.ShapeDtypeStruct((NC, n_pad), f32),
            jax.ShapeDtypeStruct((NC, n_pad), f32),
            jax.ShapeDtypeStruct((NC * e,), f32),
        ),
        mesh=mesh,
        compiler_params=sc_params,
        scratch_types=[
            pltpu.VMEM((n_pad,), f32),
            pltpu.VMEM((CH,), i32),
            pltpu.VMEM((CH,), i32),
            pltpu.VMEM((CH,), i32),
            pltpu.VMEM((CH,), i32),
            pltpu.VMEM((CH,), i32),
            pltpu.VMEM((CH,), i32),
            pltpu.VMEM((CH,), f32),
            pltpu.VMEM((CH,), f32),
            pltpu.VMEM((CH,), f32),
            pltpu.VMEM((CH,), f32),
            pltpu.VMEM((CH,), f32),
            pltpu.VMEM((CH,), f32),
            pltpu.VMEM((CH,), f32),
            pltpu.VMEM((CH,), f32),
            pltpu.SemaphoreType.DMA,
            pltpu.SemaphoreType.DMA,
            pltpu.SemaphoreType.DMA,
            pltpu.VMEM_SHARED((n_pad,), f32),
            pltpu.VMEM_SHARED((n_pad,), f32),
        ],
    )
    s2, ic, _t, _r = sc_kernel(x0, a0p, src_f, dst_f, ea_f)

    tc_rows = n_pad // 128
    out = pl.pallas_call(
        _combine2_body,
        out_shape=jax.ShapeDtypeStruct((tc_rows, 128), f32),
    )(s2.reshape(NC, tc_rows, 128), ic.reshape(NC, tc_rows, 128))
    return out.reshape(n_pad)[:n]
